# Initial kernel scaffold; baseline (speedup 1.0000x reference)
import jax
import jax.numpy as jnp
from jax.experimental import pallas as pl


def kernel(sequence, segment_label, token_table, segment_table, pe):
    raise NotImplementedError("write your pallas kernel here")


# speedup vs baseline: 5.3026x; 5.3026x over previous
import functools

import jax
import jax.numpy as jnp
from jax import lax
from jax.experimental import pallas as pl
from jax.experimental.pallas import tpu as pltpu
from jax.experimental.pallas import tpu_sc as plsc

B, S, D = 1024, 200, 128
N = B * S
NC, NS = 2, 16
NW = NC * NS
TOK_PER_W = N // NW
CH = 128
NCH = TOK_PER_W // CH


def _body(seq_hbm, lab_hbm, tok_hbm, comb_hbm, out_hbm,
          seq_v, lab_v, cidx_v, tok_rows, comb_rows, sem1, sem2):
    wid = lax.axis_index("s") * NC + lax.axis_index("c")

    def chunk_body(i, carry):
        base = wid * TOK_PER_W + i * CH
        pltpu.sync_copy(seq_hbm.at[pl.ds(base, CH)], seq_v)
        pltpu.sync_copy(lab_hbm.at[pl.ds(base, CH)], lab_v)
        for g in range(CH // 16):
            sl = pl.ds(g * 16, 16)
            pos = lax.rem(base + g * 16 + lax.iota(jnp.int32, 16), S)
            cidx_v[sl] = lab_v[sl] * S + pos
        cp1 = pltpu.async_copy(tok_hbm.at[seq_v], tok_rows, sem1)
        cp2 = pltpu.async_copy(comb_hbm.at[cidx_v], comb_rows, sem2)
        cp1.wait()
        cp2.wait()

        def row_body(r, c2):
            for g in range(D // 16):
                sl = pl.ds(g * 16, 16)
                tok_rows[r, sl] += comb_rows[r, sl]
            return c2

        lax.fori_loop(0, CH, row_body, 0)
        pltpu.sync_copy(tok_rows, out_hbm.at[pl.ds(base, CH)])
        return carry

    lax.fori_loop(0, NCH, chunk_body, 0)


@functools.partial(
    pl.kernel,
    out_type=jax.ShapeDtypeStruct((N, D), jnp.float32),
    mesh=plsc.VectorSubcoreMesh(core_axis_name="c", subcore_axis_name="s"),
    scratch_types=[
        pltpu.VMEM((CH,), jnp.int32),
        pltpu.VMEM((CH,), jnp.int32),
        pltpu.VMEM((CH,), jnp.int32),
        pltpu.VMEM((CH, D), jnp.float32),
        pltpu.VMEM((CH, D), jnp.float32),
        pltpu.SemaphoreType.DMA,
        pltpu.SemaphoreType.DMA,
    ],
)
def _sc_embed(*args):
    _body(*args)


def kernel(sequence, segment_label, token_table, segment_table, pe):
    comb = (segment_table[:, None, :] + pe[0, :S, :][None, :, :]).reshape(3 * S, D)
    out = _sc_embed(sequence.reshape(N), segment_label.reshape(N),
                    token_table, comb)
    return out.reshape(B, S, D)


# speedup vs baseline: 8.0376x; 1.5158x over previous
import functools

import jax
import jax.numpy as jnp
from jax import lax
from jax.experimental import pallas as pl
from jax.experimental.pallas import tpu as pltpu
from jax.experimental.pallas import tpu_sc as plsc

B, S, D = 1024, 200, 128
N = B * S
NC, NS = 2, 16
NW = NC * NS
TOK_PER_W = N // NW
CH = 80
NCH = TOK_PER_W // CH
NBUF = 4
NITER = NCH // NBUF


def _body(seq_hbm, lab_hbm, tok_hbm, comb_hbm, out_hbm, *rest):
    seq_v = rest[0:4]
    lab_v = rest[4:8]
    cidx_v = rest[8:12]
    tok_rows = rest[12:16]
    comb_rows = rest[16:20]
    isem = rest[20:24]
    gts = rest[24:28]
    gcs = rest[28:32]
    ws = rest[32:36]

    wid = lax.axis_index("s") * NC + lax.axis_index("c")
    w0 = wid * TOK_PER_W

    def fire_idx(b, c):
        base = w0 + c * CH
        pltpu.async_copy(seq_hbm.at[pl.ds(base, CH)], seq_v[b], isem[b])
        pltpu.async_copy(lab_hbm.at[pl.ds(base, CH)], lab_v[b], isem[b])

    def fire_gather(b, c):
        base = w0 + c * CH
        pltpu.make_async_copy(seq_hbm.at[pl.ds(0, CH)], seq_v[b], isem[b]).wait()
        pltpu.make_async_copy(lab_hbm.at[pl.ds(0, CH)], lab_v[b], isem[b]).wait()
        for g in range(CH // 16):
            sl = pl.ds(g * 16, 16)
            pos = lax.rem(base + g * 16 + lax.iota(jnp.int32, 16), S)
            cidx_v[b][sl] = lab_v[b][sl] * S + pos
        pltpu.async_copy(tok_hbm.at[seq_v[b]], tok_rows[b], gts[b])
        pltpu.async_copy(comb_hbm.at[cidx_v[b]], comb_rows[b], gcs[b])

    def consume(b, c):
        base = w0 + c * CH
        pltpu.make_async_copy(tok_hbm.at[seq_v[b]], tok_rows[b], gts[b]).wait()
        pltpu.make_async_copy(comb_hbm.at[cidx_v[b]], comb_rows[b], gcs[b]).wait()

        def row_body(r, acc):
            for g in range(D // 16):
                sl = pl.ds(g * 16, 16)
                plsc.addupdate(tok_rows[b].at[r, sl], comb_rows[b][r, sl])
            return acc

        lax.fori_loop(0, CH, row_body, 0)
        pltpu.async_copy(tok_rows[b], out_hbm.at[pl.ds(base, CH)], ws[b])

    def wait_wb(b):
        pltpu.make_async_copy(tok_rows[b], out_hbm.at[pl.ds(0, CH)], ws[b]).wait()

    for c in range(3):
        fire_idx(c % NBUF, c)
    for c in range(2):
        fire_gather(c % NBUF, c)

    def step(i, carry):
        for j in range(NBUF):
            c = i * NBUF + j
            bf2 = (j + 2) % NBUF
            bf3 = (j + 3) % NBUF
            if j >= 2:
                wait_wb(bf2)
            else:
                pl.when(i >= 1)(lambda bb=bf2: wait_wb(bb))
            if j == 0:
                fire_idx(bf3, c + 3)
            else:
                pl.when(i < NITER - 1)(lambda bb=bf3, cc=c + 3: fire_idx(bb, cc))
            if j < 2:
                fire_gather(bf2, c + 2)
            else:
                pl.when(i < NITER - 1)(lambda bb=bf2, cc=c + 2: fire_gather(bb, cc))
            consume(j, c)
        return carry

    lax.fori_loop(0, NITER, step, 0)
    wait_wb((NCH - 2) % NBUF)
    wait_wb((NCH - 1) % NBUF)


@functools.partial(
    pl.kernel,
    out_type=jax.ShapeDtypeStruct((N, D), jnp.float32),
    mesh=plsc.VectorSubcoreMesh(core_axis_name="c", subcore_axis_name="s"),
    scratch_types=(
        [pltpu.VMEM((CH,), jnp.int32) for _ in range(NBUF)]
        + [pltpu.VMEM((CH,), jnp.int32) for _ in range(NBUF)]
        + [pltpu.VMEM((CH,), jnp.int32) for _ in range(NBUF)]
        + [pltpu.VMEM((CH, D), jnp.float32) for _ in range(NBUF)]
        + [pltpu.VMEM((CH, D), jnp.float32) for _ in range(NBUF)]
        + [pltpu.SemaphoreType.DMA for _ in range(4 * NBUF)]
    ),
)
def _sc_embed(*args):
    _body(*args)


def kernel(sequence, segment_label, token_table, segment_table, pe):
    comb = (segment_table[:, None, :] + pe[0, :S, :][None, :, :]).reshape(3 * S, D)
    out = _sc_embed(sequence.reshape(N), segment_label.reshape(N),
                    token_table, comb)
    return out.reshape(B, S, D)
